# baseline (speedup 1.0000x reference)
import functools

import jax
import jax.numpy as jnp
from jax import lax
from jax.experimental import pallas as pl
from jax.experimental.pallas import tpu as pltpu
from jax.experimental.pallas import tpu_sc as plsc

_BINS = 10
_ALPHA = 0.75
_N = 4000000
_L = 16
_NC = 2
_NS = 16
_NW = _NC * _NS
_CHUNK = 8000
_NCHUNKS = _N // _CHUNK
_VPC = _CHUNK // _L
_BASE_CH = _NCHUNKS // _NW
_EXTRA = _NCHUNKS - _BASE_CH * _NW

_mesh = plsc.VectorSubcoreMesh(core_axis_name="c", subcore_axis_name="s")


@functools.partial(
    pl.kernel,
    out_type=(
        jax.ShapeDtypeStruct((_NW * _BINS * _L,), jnp.float32),
        jax.ShapeDtypeStruct((_NW * _BINS * _L,), jnp.float32),
    ),
    mesh=_mesh,
    scratch_types=[
        pltpu.VMEM((_CHUNK,), jnp.float32),
        pltpu.VMEM((_CHUNK,), jnp.float32),
        pltpu.VMEM((_CHUNK,), jnp.int32),
        pltpu.VMEM((_BINS * _L,), jnp.float32),
        pltpu.VMEM((_BINS * _L,), jnp.float32),
        pltpu.SemaphoreType.DMA,
    ],
    compiler_params=pltpu.CompilerParams(needs_layout_passes=False),
)
def _sc_hist(x1h, x2h, th, cnt_out, sum_out, x1b, x2b, tb, cacc, sacc, sem):
    wid = lax.axis_index("s") * _NC + lax.axis_index("c")
    nmine = _BASE_CH + jnp.where(wid < _EXTRA, 1, 0)
    first = wid * _BASE_CH + jnp.minimum(wid, _EXTRA)

    zeros = jnp.zeros((_L,), jnp.float32)
    for b in range(_BINS):
        cacc[pl.ds(b * _L, _L)] = zeros
        sacc[pl.ds(b * _L, _L)] = zeros

    lanes = lax.iota(jnp.int32, _L)
    ones = jnp.ones((_L,), jnp.float32)

    def chunk_body(j, carry):
        base = (first + j) * _CHUNK
        c1 = pltpu.async_copy(x1h.at[pl.ds(base, _CHUNK)], x1b, sem)
        c2 = pltpu.async_copy(x2h.at[pl.ds(base, _CHUNK)], x2b, sem)
        c3 = pltpu.async_copy(th.at[pl.ds(base, _CHUNK)], tb, sem)
        c1.wait()
        c2.wait()
        c3.wait()

        def vec_body(v, c):
            o = v * _L
            x1 = x1b[pl.ds(o, _L)]
            x2 = x2b[pl.ds(o, _L)]
            tf = tb[pl.ds(o, _L)].astype(jnp.float32)
            diff = x1 - x2
            loss = tf * jnp.maximum(-diff, 0.0)
            z = diff * (1.0 - 2.0 * tf)
            g = 1.0 / (1.0 + jnp.exp(-z))
            bin_idx = jnp.minimum((g * float(_BINS)).astype(jnp.int32), _BINS - 1)
            slot = bin_idx * _L + lanes
            plsc.addupdate_scatter(sacc, [slot], loss)
            plsc.addupdate_scatter(cacc, [slot], ones, mask=g < 1.0)
            return c

        return lax.fori_loop(0, _VPC, vec_body, carry)

    lax.fori_loop(0, nmine, chunk_body, 0)

    col = wid * (_BINS * _L)
    pltpu.sync_copy(cacc, cnt_out.at[pl.ds(col, _BINS * _L)])
    pltpu.sync_copy(sacc, sum_out.at[pl.ds(col, _BINS * _L)])


def _fin_body(cnt_ref, sum_ref, out_ref):
    tot = jnp.sum(cnt_ref[...], axis=(0, 2))
    lsum = jnp.sum(sum_ref[...], axis=(0, 2))
    tot = jnp.maximum(tot, 1.0)
    w = jnp.exp(jnp.log(tot) * (-_ALPHA))
    res = jnp.sum(w * lsum) * (1.0 / _N)
    out_ref[...] = jnp.reshape(res, (1, 1))


_finish = pl.pallas_call(
    _fin_body,
    out_shape=jax.ShapeDtypeStruct((1, 1), jnp.float32),
)


@jax.jit
def kernel(output1, output2, target):
    cnt, lsum = _sc_hist(output1, output2, target)
    cnt = cnt.reshape(_NW, _BINS, _L)
    lsum = lsum.reshape(_NW, _BINS, _L)
    return _finish(cnt, lsum)[0, 0]


# speedup vs baseline: 1.1147x; 1.1147x over previous
import functools

import jax
import jax.numpy as jnp
from jax import lax
from jax.experimental import pallas as pl
from jax.experimental.pallas import tpu as pltpu
from jax.experimental.pallas import tpu_sc as plsc

_BINS = 10
_ALPHA = 0.75
_N = 4000000
_L = 16
_NC = 2
_NS = 16
_NW = _NC * _NS
_CHUNK = 8000
_NCHUNKS = _N // _CHUNK
_VPC = _CHUNK // _L
_U = 5
_BASE_CH = _NCHUNKS // _NW
_EXTRA = _NCHUNKS - _BASE_CH * _NW
_ACC = _BINS * _L

_mesh = plsc.VectorSubcoreMesh(core_axis_name="c", subcore_axis_name="s")


@functools.partial(
    pl.kernel,
    out_type=(
        jax.ShapeDtypeStruct((_NW * _ACC,), jnp.float32),
        jax.ShapeDtypeStruct((_NW * _ACC,), jnp.float32),
    ),
    mesh=_mesh,
    scratch_types=[
        pltpu.VMEM((2 * _CHUNK,), jnp.float32),
        pltpu.VMEM((2 * _CHUNK,), jnp.float32),
        pltpu.VMEM((2 * _CHUNK,), jnp.int32),
        pltpu.VMEM((_ACC,), jnp.float32),
        pltpu.VMEM((_ACC,), jnp.float32),
        pltpu.SemaphoreType.DMA,
    ],
    compiler_params=pltpu.CompilerParams(needs_layout_passes=False),
)
def _sc_hist(x1h, x2h, th, cnt_out, sum_out, x1b, x2b, tb, cacc, sacc, sem):
    wid = lax.axis_index("s") * _NC + lax.axis_index("c")
    nmine = _BASE_CH + jnp.where(wid < _EXTRA, 1, 0)
    first = wid * _BASE_CH + jnp.minimum(wid, _EXTRA)

    zeros = jnp.zeros((_L,), jnp.float32)
    for b in range(_BINS):
        cacc[pl.ds(b * _L, _L)] = zeros
        sacc[pl.ds(b * _L, _L)] = zeros

    lanes = lax.iota(jnp.int32, _L)
    ones = jnp.ones((_L,), jnp.float32)

    def start_load(slot, ci):
        base = ci * _CHUNK
        dst = slot * _CHUNK
        pltpu.async_copy(x1h.at[pl.ds(base, _CHUNK)], x1b.at[pl.ds(dst, _CHUNK)], sem)
        pltpu.async_copy(x2h.at[pl.ds(base, _CHUNK)], x2b.at[pl.ds(dst, _CHUNK)], sem)
        pltpu.async_copy(th.at[pl.ds(base, _CHUNK)], tb.at[pl.ds(dst, _CHUNK)], sem)

    def wait_loads(slot):
        dst = slot * _CHUNK
        pltpu.make_async_copy(x1h.at[pl.ds(0, _CHUNK)], x1b.at[pl.ds(dst, _CHUNK)], sem).wait()
        pltpu.make_async_copy(x2h.at[pl.ds(0, _CHUNK)], x2b.at[pl.ds(dst, _CHUNK)], sem).wait()
        pltpu.make_async_copy(th.at[pl.ds(0, _CHUNK)], tb.at[pl.ds(dst, _CHUNK)], sem).wait()

    start_load(0, first)

    def chunk_body(j, carry):
        slot = lax.rem(j, 2)

        @pl.when(j + 1 < nmine)
        def _():
            start_load(lax.rem(j + 1, 2), first + j + 1)

        wait_loads(slot)

        sbase = slot * _CHUNK

        def blk_body(v, c):
            o = sbase + v * (_U * _L)
            for u in range(_U):
                o2 = o + u * _L
                x1 = x1b[pl.ds(o2, _L)]
                x2 = x2b[pl.ds(o2, _L)]
                tf = tb[pl.ds(o2, _L)].astype(jnp.float32)
                diff = x1 - x2
                loss = tf * jnp.maximum(-diff, 0.0)
                e = jnp.exp(diff * (2.0 * tf - 1.0))
                g10 = 10.0 / (1.0 + e)
                bi = jnp.minimum(g10.astype(jnp.int32), _BINS - 1)
                pos = bi * _L + lanes
                plsc.addupdate_scatter(sacc, [pos], loss)
                plsc.addupdate_scatter(cacc, [pos], ones, mask=g10 < 10.0)
            return c

        return lax.fori_loop(0, _VPC // _U, blk_body, carry)

    lax.fori_loop(0, nmine, chunk_body, 0)

    col = wid * _ACC
    pltpu.sync_copy(cacc, cnt_out.at[pl.ds(col, _ACC)])
    pltpu.sync_copy(sacc, sum_out.at[pl.ds(col, _ACC)])


def _fin_body(cnt_ref, sum_ref, out_ref):
    tot = jnp.sum(cnt_ref[...], axis=(0, 2))
    lsum = jnp.sum(sum_ref[...], axis=(0, 2))
    tot = jnp.maximum(tot, 1.0)
    w = jnp.exp(jnp.log(tot) * (-_ALPHA))
    res = jnp.sum(w * lsum) * (1.0 / _N)
    out_ref[...] = jnp.reshape(res, (1, 1))


_finish = pl.pallas_call(
    _fin_body,
    out_shape=jax.ShapeDtypeStruct((1, 1), jnp.float32),
)


@jax.jit
def kernel(output1, output2, target):
    cnt, lsum = _sc_hist(output1, output2, target)
    cnt = cnt.reshape(_NW, _BINS, _L)
    lsum = lsum.reshape(_NW, _BINS, _L)
    return _finish(cnt, lsum)[0, 0]


# speedup vs baseline: 4.8349x; 4.3375x over previous
import functools

import jax
import jax.numpy as jnp
from jax import lax
from jax.experimental import pallas as pl
from jax.experimental.pallas import tpu as pltpu
from jax.experimental.pallas import tpu_sc as plsc

_BINS = 10
_ALPHA = 0.75
_N = 4000000
_L = 16
_NC = 2
_NS = 16
_NW = _NC * _NS
_CHUNK = 8000
_NCHUNKS = _N // _CHUNK
_VPC = _CHUNK // _L
_U = 8
_BASE_CH = _NCHUNKS // _NW
_EXTRA = _NCHUNKS - _BASE_CH * _NW
_ACC = _BINS * _L

_mesh = plsc.VectorSubcoreMesh(core_axis_name="c", subcore_axis_name="s")


@functools.partial(
    pl.kernel,
    out_type=(
        jax.ShapeDtypeStruct((_NW * _ACC,), jnp.float32),
        jax.ShapeDtypeStruct((_NW * _ACC,), jnp.float32),
    ),
    mesh=_mesh,
    scratch_types=[
        pltpu.VMEM((2 * _CHUNK,), jnp.float32),
        pltpu.VMEM((2 * _CHUNK,), jnp.float32),
        pltpu.VMEM((2 * _CHUNK,), jnp.int32),
        pltpu.VMEM((_ACC,), jnp.float32),
        pltpu.VMEM((_ACC,), jnp.float32),
        pltpu.SemaphoreType.DMA,
        pltpu.SemaphoreType.DMA,
    ],
    compiler_params=pltpu.CompilerParams(needs_layout_passes=False),
)
def _sc_hist(x1h, x2h, th, cnt_out, sum_out, x1b, x2b, tb, cacc, sacc, sem0, sem1):
    wid = lax.axis_index("s") * _NC + lax.axis_index("c")
    nmine = _BASE_CH + jnp.where(wid < _EXTRA, 1, 0)
    first = wid * _BASE_CH + jnp.minimum(wid, _EXTRA)

    zeros = jnp.zeros((_L,), jnp.float32)
    for b in range(_BINS):
        cacc[pl.ds(b * _L, _L)] = zeros
        sacc[pl.ds(b * _L, _L)] = zeros

    lanes = lax.iota(jnp.int32, _L)
    ones = jnp.ones((_L,), jnp.float32)

    def _sem(slot):
        return lax.cond(slot == 0, lambda: 0, lambda: 1)

    def start_load(slot, ci, sem):
        base = ci * _CHUNK
        dst = slot * _CHUNK
        pltpu.async_copy(x1h.at[pl.ds(base, _CHUNK)], x1b.at[pl.ds(dst, _CHUNK)], sem)
        pltpu.async_copy(x2h.at[pl.ds(base, _CHUNK)], x2b.at[pl.ds(dst, _CHUNK)], sem)
        pltpu.async_copy(th.at[pl.ds(base, _CHUNK)], tb.at[pl.ds(dst, _CHUNK)], sem)

    def wait_loads(slot, sem):
        dst = slot * _CHUNK
        pltpu.make_async_copy(x1h.at[pl.ds(0, _CHUNK)], x1b.at[pl.ds(dst, _CHUNK)], sem).wait()
        pltpu.make_async_copy(x2h.at[pl.ds(0, _CHUNK)], x2b.at[pl.ds(dst, _CHUNK)], sem).wait()
        pltpu.make_async_copy(th.at[pl.ds(0, _CHUNK)], tb.at[pl.ds(dst, _CHUNK)], sem).wait()

    start_load(0, first, sem0)

    def chunk_body(j, carry):
        slot = lax.rem(j, 2)

        @pl.when(j + 1 < nmine)
        def _():
            @pl.when(slot == 0)
            def _():
                start_load(1, first + j + 1, sem1)

            @pl.when(slot == 1)
            def _():
                start_load(0, first + j + 1, sem0)

        @pl.when(slot == 0)
        def _():
            wait_loads(0, sem0)

        @pl.when(slot == 1)
        def _():
            wait_loads(1, sem1)

        sbase = slot * _CHUNK

        @plsc.parallel_loop(0, _VPC, 1, unroll=_U)
        def vec_body(v):
            o2 = sbase + v * _L
            x1 = x1b[pl.ds(o2, _L)]
            x2 = x2b[pl.ds(o2, _L)]
            tf = tb[pl.ds(o2, _L)].astype(jnp.float32)
            diff = x1 - x2
            loss = tf * jnp.maximum(-diff, 0.0)
            e = jnp.exp(diff * (2.0 * tf - 1.0))
            g10 = 10.0 / (1.0 + e)
            bi = jnp.minimum(g10.astype(jnp.int32), _BINS - 1)
            pos = bi * _L + lanes
            plsc.addupdate_scatter(sacc, [pos], loss)
            plsc.addupdate_scatter(cacc, [pos], ones, mask=g10 < 10.0)

        return carry

    lax.fori_loop(0, nmine, chunk_body, 0)

    col = wid * _ACC
    pltpu.sync_copy(cacc, cnt_out.at[pl.ds(col, _ACC)])
    pltpu.sync_copy(sacc, sum_out.at[pl.ds(col, _ACC)])


def _fin_body(cnt_ref, sum_ref, out_ref):
    tot = jnp.sum(cnt_ref[...], axis=(0, 2))
    lsum = jnp.sum(sum_ref[...], axis=(0, 2))
    tot = jnp.maximum(tot, 1.0)
    w = jnp.exp(jnp.log(tot) * (-_ALPHA))
    res = jnp.sum(w * lsum) * (1.0 / _N)
    out_ref[...] = jnp.reshape(res, (1, 1))


_finish = pl.pallas_call(
    _fin_body,
    out_shape=jax.ShapeDtypeStruct((1, 1), jnp.float32),
)


@jax.jit
def kernel(output1, output2, target):
    cnt, lsum = _sc_hist(output1, output2, target)
    cnt = cnt.reshape(_NW, _BINS, _L)
    lsum = lsum.reshape(_NW, _BINS, _L)
    return _finish(cnt, lsum)[0, 0]
